# Initial kernel scaffold; baseline (speedup 1.0000x reference)
import jax
import jax.numpy as jnp
from jax.experimental import pallas as pl


def kernel(inputs, norm1_w, norm1_b, norm2_w, norm2_b, Wq, bq, Wk, bk, Wv, bv, Wo, bo, w_score, W1, b1, W2, b2):
    raise NotImplementedError("write your pallas kernel here")


# speedup vs baseline: 10.9444x; 10.9444x over previous
import dataclasses
import functools

import numpy as np
import jax
import jax.numpy as jnp
from jax import lax
from jax.experimental import pallas as pl
from jax.experimental.pallas import tpu as pltpu
from jax.experimental.pallas import tpu_sc as plsc

B, R, C, H, K_KNN, K_SEL, MLP = 4, 16, 64, 4, 10, 4, 256
N = R ** 3
BN = B * N
DH = C // H


def _knn_table():
    g = np.arange(R)
    coords = np.stack(np.meshgrid(g, g, g, indexing="ij"), axis=-1)
    coords = coords.reshape(N, 3).astype(np.int64)
    d2 = ((coords[:, None, :] - coords[None, :, :]) ** 2).sum(-1)
    order = np.argsort(d2, axis=1, kind="stable")[:, :K_KNN]
    kt = order.T.astype(np.int64)
    cols = [kt + b * N for b in range(B)]
    return np.concatenate(cols, axis=1).astype(np.int32)


_KNNT = _knn_table()

_PREC = lax.Precision.HIGHEST


def _dot(a, b):
    return lax.dot_general(a, b, (((1,), (0,)), ((), ())),
                           preferred_element_type=jnp.float32,
                           precision=_PREC)


_TA = 2048


def _stage_a_body(x_ref, n1w_ref, n1b_ref, wq_ref, bq_ref, wkv_ref, ws_ref,
                  q_ref, kv_ref, s_ref):
    x = x_ref[...]
    m = jnp.mean(x, axis=1, keepdims=True)
    v = jnp.mean((x - m) ** 2, axis=1, keepdims=True)
    ln = (x - m) / jnp.sqrt(v + 1e-5) * n1w_ref[...] + n1b_ref[...]
    q_ref[...] = _dot(ln, wq_ref[...]) + bq_ref[...]
    kv_ref[...] = _dot(ln, wkv_ref[...])
    s_ref[...] = jnp.sum(ln * ws_ref[...], axis=1, keepdims=True)


def _stage_a(x, n1w, n1b, wq, bq, wkv, ws):
    nblk = BN // _TA
    full = lambda shape: pl.BlockSpec(shape, lambda i: (0, 0))
    return pl.pallas_call(
        _stage_a_body,
        grid=(nblk,),
        in_specs=[
            pl.BlockSpec((_TA, C), lambda i: (i, 0)),
            full((1, C)), full((1, C)),
            full((C, C)), full((1, C)),
            full((C, 2 * C)), full((1, C)),
        ],
        out_specs=[
            pl.BlockSpec((_TA, C), lambda i: (i, 0)),
            pl.BlockSpec((_TA, 2 * C), lambda i: (i, 0)),
            pl.BlockSpec((_TA, 1), lambda i: (i, 0)),
        ],
        out_shape=[
            jax.ShapeDtypeStruct((BN, C), jnp.float32),
            jax.ShapeDtypeStruct((BN, 2 * C), jnp.float32),
            jax.ShapeDtypeStruct((BN, 1), jnp.float32),
        ],
    )(x, n1w, n1b, wq, bq, wkv, ws)


_NW = 32
_TPW = BN // _NW
_CH = 64
_NCH = _TPW // _CH
_GPC = _CH // 16

_NEG = -3.4e38


def _sc_body(s_hbm, knn_hbm, xkv_hbm,
             kv0_hbm, kv1_hbm, kv2_hbm, kv3_hbm, gates_hbm,
             s_v, knn_v, gid_v, rows_v, gates_v, semg):
    kv_outs = (kv0_hbm, kv1_hbm, kv2_hbm, kv3_hbm)
    w = lax.axis_index("s") * 2 + lax.axis_index("c")
    wbase = w * _TPW
    pltpu.sync_copy(s_hbm, s_v)
    pltpu.sync_copy(knn_hbm.at[:, pl.ds(wbase, _TPW)], knn_v)

    lane = lax.iota(jnp.int32, 16)

    for c in range(_NCH):
        @pl.loop(0, _GPC)
        def _(g, c=c):
            lb = c * _CH + g * 16
            cand = []
            gids = []
            for k in range(K_KNN):
                idx_k = knn_v[k, pl.ds(lb, 16)]
                gids.append(idx_k)
                cand.append(plsc.load_gather(s_v, [idx_k]))
            for j in range(K_SEL):
                m = cand[0]
                for k in range(1, K_KNN):
                    m = jnp.maximum(m, cand[k])
                found = lane < 0
                chosen = gids[0]
                for k in range(K_KNN):
                    eq = jnp.logical_and(cand[k] == m,
                                         jnp.logical_not(found))
                    chosen = jnp.where(eq, gids[k], chosen)
                    cand[k] = jnp.where(eq, _NEG, cand[k])
                    found = jnp.logical_or(found, eq)
                gate = 1.0 / (1.0 + jnp.exp(-m))
                gid_v[j, pl.ds(lb, 16)] = chosen
                plsc.store_scatter(gates_v, [lb + lane,
                                             jnp.full((16,), j, jnp.int32)],
                                   gate)

        copies = []
        for j in range(K_SEL):
            copies.append(pltpu.async_copy(
                xkv_hbm.at[gid_v.at[j, pl.ds(c * _CH, _CH)]],
                rows_v.at[j], semg))
        for cp in copies:
            cp.wait()
        for j in range(K_SEL):
            pltpu.sync_copy(rows_v.at[j],
                            kv_outs[j].at[pl.ds(wbase + c * _CH, _CH)])

    pltpu.sync_copy(gates_v, gates_hbm.at[pl.ds(wbase, _TPW)])


def _stage_b(s_flat, knn, xkv):
    mesh = plsc.VectorSubcoreMesh(core_axis_name="c", subcore_axis_name="s")
    row = jax.ShapeDtypeStruct((BN, 2 * C), jnp.float32)
    cp = pltpu.CompilerParams()
    if "needs_layout_passes" in pltpu.CompilerParams.__dataclass_fields__:
        cp = dataclasses.replace(cp, needs_layout_passes=False)
    kern = functools.partial(
        pl.kernel,
        mesh=mesh,
        compiler_params=cp,
        out_type=[row, row, row, row,
                  jax.ShapeDtypeStruct((BN, K_SEL), jnp.float32)],
        scratch_types=[
            pltpu.VMEM((BN,), jnp.float32),
            pltpu.VMEM((K_KNN, _TPW), jnp.int32),
            pltpu.VMEM((K_SEL, _TPW), jnp.int32),
            pltpu.VMEM((K_SEL, _CH, 2 * C), jnp.float32),
            pltpu.VMEM((_TPW, K_SEL), jnp.float32),
            pltpu.SemaphoreType.DMA,
        ],
    )(_sc_body)
    return kern(s_flat, knn, xkv)


_TC = 1024
_NBC = BN // _TC


def _stage_c_body(q_ref, kv0_ref, kv1_ref, kv2_ref, kv3_ref, g_ref,
                  sc_ref, bk_ref, bv_ref, wo_ref, bo_ref,
                  n2w_ref, n2b_ref, w1_ref, b1_ref, w2_ref, b2_ref,
                  out_ref):
    q = q_ref[...]
    g = g_ref[0]
    kv = (kv0_ref[...], kv1_ref[...], kv2_ref[...], kv3_ref[...])

    rows = lax.broadcasted_iota(jnp.int32, (C, H), 0) // DH
    cols = lax.broadcasted_iota(jnp.int32, (C, H), 1)
    S = (rows == cols).astype(jnp.float32)
    rows_t = lax.broadcasted_iota(jnp.int32, (H, C), 0)
    cols_t = lax.broadcasted_iota(jnp.int32, (H, C), 1) // DH
    ST = (rows_t == cols_t).astype(jnp.float32)

    iota4 = lax.broadcasted_iota(jnp.int32, (1, H), 1)
    qbk = _dot(q * bk_ref[...], S)

    scale = jnp.float32(1.0) / jnp.sqrt(jnp.float32(DH))
    g_cols = []
    logits = []
    for j in range(K_SEL):
        g_j = jnp.sum(jnp.where(iota4 == j, g, 0.0), axis=1, keepdims=True)
        g_cols.append(g_j)
        hs = _dot(q * kv[j][:, :C], S)
        logits.append((hs * g_j + qbk) * scale)

    m = jnp.maximum(jnp.maximum(logits[0], logits[1]),
                    jnp.maximum(logits[2], logits[3]))
    es = [jnp.exp(l - m) for l in logits]
    z = es[0] + es[1] + es[2] + es[3]

    out = jnp.zeros_like(q)
    for j in range(K_SEL):
        att_e = _dot(es[j] / z, ST)
        out = out + att_e * (kv[j][:, C:] * g_cols[j] + bv_ref[...])

    o = _dot(out, wo_ref[...]) + bo_ref[...]
    x1 = o * 0.5 + sc_ref[...]

    mu = jnp.mean(x1, axis=1, keepdims=True)
    var = jnp.mean((x1 - mu) ** 2, axis=1, keepdims=True)
    y = (x1 - mu) / jnp.sqrt(var + 1e-5) * n2w_ref[...] + n2b_ref[...]
    h = jax.nn.gelu(_dot(y, w1_ref[...]) + b1_ref[...])
    y2 = _dot(h, w2_ref[...]) + b2_ref[...]
    out_ref[...] = y2 * 0.5 + x1


def _stage_c(q, kvs, gates3, shortcut, bk, bv, wo, bo, n2w, n2b, w1, b1, w2, b2):
    full = lambda shape: pl.BlockSpec(shape, lambda i: tuple(0 for _ in shape))
    tok = lambda width: pl.BlockSpec((_TC, width), lambda i: (i, 0))
    return pl.pallas_call(
        _stage_c_body,
        grid=(_NBC,),
        in_specs=[
            tok(C),
            tok(2 * C), tok(2 * C), tok(2 * C), tok(2 * C),
            pl.BlockSpec((1, _TC, K_SEL), lambda i: (i, 0, 0)),
            tok(C),
            full((1, C)), full((1, C)),
            full((C, C)), full((1, C)),
            full((1, C)), full((1, C)),
            full((C, MLP)), full((1, MLP)),
            full((MLP, C)), full((1, C)),
        ],
        out_specs=pl.BlockSpec((_TC, C), lambda i: (i, 0)),
        out_shape=jax.ShapeDtypeStruct((BN, C), jnp.float32),
    )(q, *kvs, gates3, shortcut, bk, bv, wo, bo, n2w, n2b, w1, b1, w2, b2)


def kernel(inputs, norm1_w, norm1_b, norm2_w, norm2_b, Wq, bq, Wk, bk, Wv, bv,
           Wo, bo, w_score, W1, b1, W2, b2):
    x = inputs.reshape(BN, C)
    wkv = jnp.concatenate([Wk, Wv], axis=1)
    row = lambda a: a.reshape(1, -1)

    q, xkv, s = _stage_a(x, row(norm1_w), row(norm1_b), Wq, row(bq), wkv,
                         row(w_score))

    knn = jnp.asarray(_KNNT)
    kv0, kv1, kv2, kv3, gates = _stage_b(s.reshape(BN), knn, xkv)

    y = _stage_c(q, (kv0, kv1, kv2, kv3), gates.reshape(_NBC, _TC, K_SEL), x,
                 row(bk), row(bv), Wo, row(bo), row(norm2_w), row(norm2_b),
                 W1, row(b1), W2, row(b2))
    return y.reshape(B, N, C)


# speedup vs baseline: 18.0154x; 1.6461x over previous
import dataclasses
import functools

import numpy as np
import jax
import jax.numpy as jnp
from jax import lax
from jax.experimental import pallas as pl
from jax.experimental.pallas import tpu as pltpu
from jax.experimental.pallas import tpu_sc as plsc

B, R, C, H, K_KNN, K_SEL, MLP = 4, 16, 64, 4, 10, 4, 256
N = R ** 3
BN = B * N
DH = C // H


def _knn_table():
    g = np.arange(R)
    coords = np.stack(np.meshgrid(g, g, g, indexing="ij"), axis=-1)
    coords = coords.reshape(N, 3).astype(np.int64)
    d2 = ((coords[:, None, :] - coords[None, :, :]) ** 2).sum(-1)
    order = np.argsort(d2, axis=1, kind="stable")[:, :K_KNN]
    kt = order.T.astype(np.int64)
    cols = [kt + b * N for b in range(B)]
    return np.concatenate(cols, axis=1).astype(np.int32)


_KNNT = _knn_table()

_PREC = lax.Precision.DEFAULT


def _dot(a, b):
    return lax.dot_general(a, b, (((1,), (0,)), ((), ())),
                           preferred_element_type=jnp.float32,
                           precision=_PREC)


_TA = 2048


def _stage_a_body(x_ref, n1w_ref, n1b_ref, wq_ref, bq_ref, wkv_ref, ws_ref,
                  q_ref, kv_ref, s_ref):
    x = x_ref[...]
    m = jnp.mean(x, axis=1, keepdims=True)
    v = jnp.mean((x - m) ** 2, axis=1, keepdims=True)
    ln = (x - m) / jnp.sqrt(v + 1e-5) * n1w_ref[...] + n1b_ref[...]
    q_ref[...] = _dot(ln, wq_ref[...]) + bq_ref[...]
    kv_ref[...] = _dot(ln, wkv_ref[...])
    s_ref[...] = jnp.sum(ln * ws_ref[...], axis=1, keepdims=True)


def _stage_a(x, n1w, n1b, wq, bq, wkv, ws):
    nblk = BN // _TA
    full = lambda shape: pl.BlockSpec(shape, lambda i: (0, 0))
    return pl.pallas_call(
        _stage_a_body,
        grid=(nblk,),
        in_specs=[
            pl.BlockSpec((_TA, C), lambda i: (i, 0)),
            full((1, C)), full((1, C)),
            full((C, C)), full((1, C)),
            full((C, 2 * C)), full((1, C)),
        ],
        out_specs=[
            pl.BlockSpec((_TA, C), lambda i: (i, 0)),
            pl.BlockSpec((_TA, 2 * C), lambda i: (i, 0)),
            pl.BlockSpec((_TA, 1), lambda i: (i, 0)),
        ],
        out_shape=[
            jax.ShapeDtypeStruct((BN, C), jnp.float32),
            jax.ShapeDtypeStruct((BN, 2 * C), jnp.float32),
            jax.ShapeDtypeStruct((BN, 1), jnp.float32),
        ],
    )(x, n1w, n1b, wq, bq, wkv, ws)


_NW = 32
_TPW = BN // _NW
_CH = 64
_NCH = _TPW // _CH
_GPC = _CH // 16

_NEG = -3.4e38


def _sc_body(s_hbm, knn_hbm, xkv_hbm,
             kv0_hbm, kv1_hbm, kv2_hbm, kv3_hbm, gates_hbm,
             s_v, knn_v, gid_v, rows_v, gates_v, semg):
    kv_outs = (kv0_hbm, kv1_hbm, kv2_hbm, kv3_hbm)
    w = lax.axis_index("s") * 2 + lax.axis_index("c")
    wbase = w * _TPW
    pltpu.sync_copy(s_hbm, s_v)
    pltpu.sync_copy(knn_hbm.at[:, pl.ds(wbase, _TPW)], knn_v)

    lane = lax.iota(jnp.int32, 16)

    for c in range(_NCH):
        @pl.loop(0, _GPC)
        def _(g, c=c):
            lb = c * _CH + g * 16
            cand = []
            gids = []
            for k in range(K_KNN):
                idx_k = knn_v[k, pl.ds(lb, 16)]
                gids.append(idx_k)
                cand.append(plsc.load_gather(s_v, [idx_k]))
            for j in range(K_SEL):
                m = cand[0]
                for k in range(1, K_KNN):
                    m = jnp.maximum(m, cand[k])
                found = lane < 0
                chosen = gids[0]
                for k in range(K_KNN):
                    eq = jnp.logical_and(cand[k] == m,
                                         jnp.logical_not(found))
                    chosen = jnp.where(eq, gids[k], chosen)
                    cand[k] = jnp.where(eq, _NEG, cand[k])
                    found = jnp.logical_or(found, eq)
                gate = 1.0 / (1.0 + jnp.exp(-m))
                gid_v[j, pl.ds(lb, 16)] = chosen
                plsc.store_scatter(gates_v, [lb + lane,
                                             jnp.full((16,), j, jnp.int32)],
                                   gate)

        copies = []
        for j in range(K_SEL):
            copies.append(pltpu.async_copy(
                xkv_hbm.at[gid_v.at[j, pl.ds(c * _CH, _CH)]],
                rows_v.at[j], semg))
        for cp in copies:
            cp.wait()
        for j in range(K_SEL):
            pltpu.sync_copy(rows_v.at[j],
                            kv_outs[j].at[pl.ds(wbase + c * _CH, _CH)])

    pltpu.sync_copy(gates_v, gates_hbm.at[pl.ds(wbase, _TPW)])


def _stage_b(s_flat, knn, xkv):
    mesh = plsc.VectorSubcoreMesh(core_axis_name="c", subcore_axis_name="s")
    row = jax.ShapeDtypeStruct((BN, 2 * C), jnp.float32)
    cp = pltpu.CompilerParams()
    if "needs_layout_passes" in pltpu.CompilerParams.__dataclass_fields__:
        cp = dataclasses.replace(cp, needs_layout_passes=False)
    kern = functools.partial(
        pl.kernel,
        mesh=mesh,
        compiler_params=cp,
        out_type=[row, row, row, row,
                  jax.ShapeDtypeStruct((BN, K_SEL), jnp.float32)],
        scratch_types=[
            pltpu.VMEM((BN,), jnp.float32),
            pltpu.VMEM((K_KNN, _TPW), jnp.int32),
            pltpu.VMEM((K_SEL, _TPW), jnp.int32),
            pltpu.VMEM((K_SEL, _CH, 2 * C), jnp.float32),
            pltpu.VMEM((_TPW, K_SEL), jnp.float32),
            pltpu.SemaphoreType.DMA,
        ],
    )(_sc_body)
    return kern(s_flat, knn, xkv)


_TC = 1024
_NBC = BN // _TC


def _stage_c_body(q_ref, kv0_ref, kv1_ref, kv2_ref, kv3_ref, g_ref,
                  sc_ref, bk_ref, bv_ref, wo_ref, bo_ref,
                  n2w_ref, n2b_ref, w1_ref, b1_ref, w2_ref, b2_ref,
                  out_ref):
    q = q_ref[...]
    g = g_ref[0]
    kv = (kv0_ref[...], kv1_ref[...], kv2_ref[...], kv3_ref[...])

    rows = lax.broadcasted_iota(jnp.int32, (C, H), 0) // DH
    cols = lax.broadcasted_iota(jnp.int32, (C, H), 1)
    S = (rows == cols).astype(jnp.float32)
    rows_t = lax.broadcasted_iota(jnp.int32, (H, C), 0)
    cols_t = lax.broadcasted_iota(jnp.int32, (H, C), 1) // DH
    ST = (rows_t == cols_t).astype(jnp.float32)

    iota4 = lax.broadcasted_iota(jnp.int32, (1, H), 1)
    qbk = _dot(q * bk_ref[...], S)

    scale = jnp.float32(1.0) / jnp.sqrt(jnp.float32(DH))
    g_cols = []
    logits = []
    for j in range(K_SEL):
        g_j = jnp.sum(jnp.where(iota4 == j, g, 0.0), axis=1, keepdims=True)
        g_cols.append(g_j)
        hs = _dot(q * kv[j][:, :C], S)
        logits.append((hs * g_j + qbk) * scale)

    m = jnp.maximum(jnp.maximum(logits[0], logits[1]),
                    jnp.maximum(logits[2], logits[3]))
    es = [jnp.exp(l - m) for l in logits]
    z = es[0] + es[1] + es[2] + es[3]

    out = jnp.zeros_like(q)
    for j in range(K_SEL):
        att_e = _dot(es[j] / z, ST)
        out = out + att_e * (kv[j][:, C:] * g_cols[j] + bv_ref[...])

    o = _dot(out, wo_ref[...]) + bo_ref[...]
    x1 = o * 0.5 + sc_ref[...]

    mu = jnp.mean(x1, axis=1, keepdims=True)
    var = jnp.mean((x1 - mu) ** 2, axis=1, keepdims=True)
    y = (x1 - mu) / jnp.sqrt(var + 1e-5) * n2w_ref[...] + n2b_ref[...]
    h = jax.nn.gelu(_dot(y, w1_ref[...]) + b1_ref[...])
    y2 = _dot(h, w2_ref[...]) + b2_ref[...]
    out_ref[...] = y2 * 0.5 + x1


def _stage_c(q, kvs, gates3, shortcut, bk, bv, wo, bo, n2w, n2b, w1, b1, w2, b2):
    full = lambda shape: pl.BlockSpec(shape, lambda i: tuple(0 for _ in shape))
    tok = lambda width: pl.BlockSpec((_TC, width), lambda i: (i, 0))
    return pl.pallas_call(
        _stage_c_body,
        grid=(_NBC,),
        in_specs=[
            tok(C),
            tok(2 * C), tok(2 * C), tok(2 * C), tok(2 * C),
            pl.BlockSpec((1, _TC, K_SEL), lambda i: (i, 0, 0)),
            tok(C),
            full((1, C)), full((1, C)),
            full((C, C)), full((1, C)),
            full((1, C)), full((1, C)),
            full((C, MLP)), full((1, MLP)),
            full((MLP, C)), full((1, C)),
        ],
        out_specs=pl.BlockSpec((_TC, C), lambda i: (i, 0)),
        out_shape=jax.ShapeDtypeStruct((BN, C), jnp.float32),
    )(q, *kvs, gates3, shortcut, bk, bv, wo, bo, n2w, n2b, w1, b1, w2, b2)


def kernel(inputs, norm1_w, norm1_b, norm2_w, norm2_b, Wq, bq, Wk, bk, Wv, bv,
           Wo, bo, w_score, W1, b1, W2, b2):
    x = inputs.reshape(BN, C)
    wkv = jnp.concatenate([Wk, Wv], axis=1)
    row = lambda a: a.reshape(1, -1)

    q, xkv, s = _stage_a(x, row(norm1_w), row(norm1_b), Wq, row(bq), wkv,
                         row(w_score))

    knn = jnp.asarray(_KNNT)
    kv0, kv1, kv2, kv3, gates = _stage_b(s.reshape(BN), knn, xkv)

    y = _stage_c(q, (kv0, kv1, kv2, kv3), gates.reshape(_NBC, _TC, K_SEL), x,
                 row(bk), row(bv), Wo, row(bo), row(norm2_w), row(norm2_b),
                 W1, row(b1), W2, row(b2))
    return y.reshape(B, N, C)
